# Initial kernel scaffold; baseline (speedup 1.0000x reference)
import jax
import jax.numpy as jnp
from jax.experimental import pallas as pl


def kernel(x, edge_index, W1l, W1r, b1, W2l, W2r, b2, Wd1, bd1, Wd2, bd2, Wd3, bd3):
    raise NotImplementedError("write your pallas kernel here")


# speedup vs baseline: 4.9393x; 4.9393x over previous
import functools

import jax
import jax.numpy as jnp
from jax import lax
from jax.experimental import pallas as pl
from jax.experimental.pallas import tpu as pltpu
from jax.experimental.pallas import tpu_sc as plsc

N = 100000
E = 1600000
D = 128
C = 16
NCH = D // C
NC = 2
NS = 16
BE = 1000
RPT = N // NS

TN = 2000
IDXB = 3200


def _sc_mesh():
    return plsc.VectorSubcoreMesh(core_axis_name="c", subcore_axis_name="s")


def _zero_acc_slice(rows, acc, tbase):
    def zf(i, _):
        rows[i, :] = jnp.zeros((C,), jnp.float32)
        return 0

    lax.fori_loop(0, BE, zf, 0)
    nfull = RPT // BE
    rem = RPT % BE
    for k in range(nfull):
        pltpu.sync_copy(rows, acc.at[pl.ds(tbase + k * BE, BE)])
    if rem:
        pltpu.sync_copy(rows.at[pl.ds(0, rem)],
                        acc.at[pl.ds(tbase + nfull * BE, rem)])


def _stage_a_body(x16, src, dst, out, sidx, didx, rows, acc, sem):
    sc = lax.axis_index("c")
    tid = lax.axis_index("s")
    tbase = tid * RPT
    _zero_acc_slice(rows, acc, tbase)
    plsc.subcore_barrier()

    epc = E // NC
    ept = epc // NS
    ebase = sc * epc + tid * ept
    nb = ept // BE

    def body(bi, _):
        off = ebase + bi * BE
        pltpu.sync_copy(src.at[pl.ds(off, BE)], sidx)
        pltpu.sync_copy(dst.at[pl.ds(off, BE)], didx)
        pltpu.async_copy(x16.at[sidx], rows, sem).wait()
        pltpu.sync_copy(rows, acc.at[didx], add=True)
        return 0

    lax.fori_loop(0, nb, body, 0)
    plsc.subcore_barrier()
    pltpu.sync_copy(acc.at[pl.ds(tbase, RPT)], out.at[sc, pl.ds(tbase, RPT)])


def _stage_c_body(h1f, idxmat, dst, out, sidx, didx, rows, acc, sem):
    sc = lax.axis_index("c")
    tid = lax.axis_index("s")
    tbase = tid * RPT
    ept = E // NS
    ebase = tid * ept
    nb = ept // BE

    for cl in range(NCH // NC):
        chunk = sc * (NCH // NC) + cl
        _zero_acc_slice(rows, acc, tbase)
        plsc.subcore_barrier()

        def body(bi, _):
            off = ebase + bi * BE
            pltpu.sync_copy(idxmat.at[chunk, pl.ds(off, BE)], sidx)
            pltpu.sync_copy(dst.at[pl.ds(off, BE)], didx)
            pltpu.async_copy(h1f.at[sidx], rows, sem).wait()
            pltpu.sync_copy(rows, acc.at[didx], add=True)
            return 0

        lax.fori_loop(0, nb, body, 0)
        plsc.subcore_barrier()
        pltpu.sync_copy(acc.at[pl.ds(tbase, RPT)],
                        out.at[pl.ds(tbase, RPT), chunk])
        plsc.subcore_barrier()


def _run_stage_a(x16, src, dst):
    f = pl.kernel(
        _stage_a_body,
        out_type=jax.ShapeDtypeStruct((NC, N, C), jnp.float32),
        mesh=_sc_mesh(),
        compiler_params=pltpu.CompilerParams(use_tc_tiling_on_sc=False),
        scratch_types=[
            pltpu.VMEM((BE,), jnp.int32),
            pltpu.VMEM((BE,), jnp.int32),
            pltpu.VMEM((BE, C), jnp.float32),
            pltpu.VMEM_SHARED((N, C), jnp.float32),
            pltpu.SemaphoreType.DMA,
        ],
    )
    return f(x16, src, dst)


def _run_stage_c(h1f, idxmat, dst):
    f = pl.kernel(
        _stage_c_body,
        out_type=jax.ShapeDtypeStruct((N, NCH, C), jnp.float32),
        mesh=_sc_mesh(),
        compiler_params=pltpu.CompilerParams(use_tc_tiling_on_sc=False),
        scratch_types=[
            pltpu.VMEM((BE,), jnp.int32),
            pltpu.VMEM((BE,), jnp.int32),
            pltpu.VMEM((BE, C), jnp.float32),
            pltpu.VMEM_SHARED((N, C), jnp.float32),
            pltpu.SemaphoreType.DMA,
        ],
    )
    return f(h1f, idxmat, dst)


def _idx_body(src_ref, out_ref):
    out_ref[...] = src_ref[...] * NCH + lax.broadcasted_iota(
        jnp.int32, (NCH, IDXB), 0)


def _build_idxmat(src):
    return pl.pallas_call(
        _idx_body,
        grid=(E // IDXB,),
        in_specs=[pl.BlockSpec((1, IDXB), lambda i: (0, i))],
        out_specs=pl.BlockSpec((NCH, IDXB), lambda i: (0, i)),
        out_shape=jax.ShapeDtypeStruct((NCH, E), jnp.int32),
    )(src.reshape(1, E))


def _h1_body(x_ref, agg_ref, w1l_ref, w1r_ref, b1_ref, out_ref):
    s = agg_ref[0] + agg_ref[1]
    cnt = jnp.maximum(s[:, 2:3], 1.0)
    mean1 = s[:, 0:2] / cnt
    h = jnp.dot(x_ref[...], w1l_ref[...], preferred_element_type=jnp.float32)
    h = h + jnp.dot(mean1, w1r_ref[...], preferred_element_type=jnp.float32)
    out_ref[...] = jnp.maximum(h + b1_ref[...], 0.0)


def _run_h1(x, agg1p, W1l, W1r, b1):
    return pl.pallas_call(
        _h1_body,
        grid=(N // TN,),
        in_specs=[
            pl.BlockSpec((TN, 2), lambda i: (i, 0)),
            pl.BlockSpec((NC, TN, C), lambda i: (0, i, 0)),
            pl.BlockSpec((2, D), lambda i: (0, 0)),
            pl.BlockSpec((2, D), lambda i: (0, 0)),
            pl.BlockSpec((1, D), lambda i: (0, 0)),
        ],
        out_specs=pl.BlockSpec((TN, D), lambda i: (i, 0)),
        out_shape=jax.ShapeDtypeStruct((N, D), jnp.float32),
    )(x, agg1p, W1l, W1r, b1.reshape(1, D))


def _dec_body(h1_ref, agg2_ref, agg1_ref, w2l, w2r, b2, wd1, bdd1, wd2, bdd2,
              wd3, bdd3, out_ref):
    s = agg1_ref[0] + agg1_ref[1]
    cnt = jnp.maximum(s[:, 2:3], 1.0)
    mean2 = agg2_ref[...] / cnt
    h1 = h1_ref[...]
    h2 = jnp.dot(h1, w2l[...], preferred_element_type=jnp.float32)
    h2 = h2 + jnp.dot(mean2, w2r[...], preferred_element_type=jnp.float32)
    h2 = h2 + b2[...]
    z = jnp.maximum(
        jnp.dot(h2, wd1[...], preferred_element_type=jnp.float32) + bdd1[...],
        0.0)
    z = jnp.maximum(
        jnp.dot(z, wd2[...], preferred_element_type=jnp.float32) + bdd2[...],
        0.0)
    out_ref[...] = jnp.dot(z, wd3[...],
                           preferred_element_type=jnp.float32) + bdd3[...]


def _run_decoder(h1, agg2, agg1p, W2l, W2r, b2, Wd1, bd1, Wd2, bd2, Wd3, bd3):
    full = lambda shape: pl.BlockSpec(shape, lambda i: tuple(0 for _ in shape))
    return pl.pallas_call(
        _dec_body,
        grid=(N // TN,),
        in_specs=[
            pl.BlockSpec((TN, D), lambda i: (i, 0)),
            pl.BlockSpec((TN, D), lambda i: (i, 0)),
            pl.BlockSpec((NC, TN, C), lambda i: (0, i, 0)),
            full((D, D)),
            full((D, D)),
            full((1, D)),
            full((D, 128)),
            full((1, 128)),
            full((128, 64)),
            full((1, 64)),
            full((64, 1)),
            full((1, 1)),
        ],
        out_specs=pl.BlockSpec((TN, 1), lambda i: (i, 0)),
        out_shape=jax.ShapeDtypeStruct((N, 1), jnp.float32),
    )(h1, agg2, agg1p, W2l, W2r, b2.reshape(1, D), Wd1, bd1.reshape(1, 128),
      Wd2, bd2.reshape(1, 64), Wd3, bd3.reshape(1, 1))


def kernel(x, edge_index, W1l, W1r, b1, W2l, W2r, b2, Wd1, bd1, Wd2, bd2, Wd3,
           bd3):
    src = edge_index[0].astype(jnp.int32)
    dst = edge_index[1].astype(jnp.int32)
    x16 = jnp.zeros((N, C), jnp.float32).at[:, :2].set(x).at[:, 2].set(1.0)

    agg1p = _run_stage_a(x16, src, dst)
    idxmat = _build_idxmat(src)
    h1 = _run_h1(x, agg1p, W1l, W1r, b1)
    agg2 = _run_stage_c(h1.reshape(N * NCH, C), idxmat, dst)
    q = _run_decoder(h1, agg2.reshape(N, D), agg1p, W2l, W2r, b2, Wd1, bd1,
                     Wd2, bd2, Wd3, bd3)
    return q.reshape(N)


# speedup vs baseline: 7.0224x; 1.4217x over previous
import functools

import jax
import jax.numpy as jnp
from jax import lax
from jax.experimental import pallas as pl
from jax.experimental.pallas import tpu as pltpu
from jax.experimental.pallas import tpu_sc as plsc

N = 100000
E = 1600000
D = 128
C = 16
NCH = D // C
NC = 2
NS = 16
BE = 1000
BEC = 800
RPT = N // NS

TN = 2000
IDXB = 3200


def _sc_mesh():
    return plsc.VectorSubcoreMesh(core_axis_name="c", subcore_axis_name="s")


def _zero_acc_slice(rows, acc, tbase, nrows):
    def zf(i, _):
        rows[i, :] = jnp.zeros((C,), jnp.float32)
        return 0

    lax.fori_loop(0, nrows, zf, 0)
    nfull = RPT // nrows
    rem = RPT % nrows
    for k in range(nfull):
        pltpu.sync_copy(rows, acc.at[pl.ds(tbase + k * nrows, nrows)])
    if rem:
        pltpu.sync_copy(rows.at[pl.ds(0, rem)],
                        acc.at[pl.ds(tbase + nfull * nrows, rem)])


def _stage_a_body(x16, src, dst, out, sidx, didx, rows, acc, sem):
    sc = lax.axis_index("c")
    tid = lax.axis_index("s")
    tbase = tid * RPT
    _zero_acc_slice(rows, acc, tbase, BE)
    plsc.subcore_barrier()

    epc = E // NC
    ept = epc // NS
    ebase = sc * epc + tid * ept
    nb = ept // BE

    def body(bi, _):
        off = ebase + bi * BE
        pltpu.sync_copy(src.at[pl.ds(off, BE)], sidx)
        pltpu.sync_copy(dst.at[pl.ds(off, BE)], didx)
        pltpu.async_copy(x16.at[sidx], rows, sem).wait()
        pltpu.sync_copy(rows, acc.at[didx], add=True)
        return 0

    lax.fori_loop(0, nb, body, 0)
    plsc.subcore_barrier()
    pltpu.sync_copy(acc.at[pl.ds(tbase, RPT)], out.at[sc, pl.ds(tbase, RPT)])


def _stage_c_body(h1f, src, dst, out, sidx, didx, gidx, rows, acc, sem):
    sc = lax.axis_index("c")
    tid = lax.axis_index("s")
    tbase = tid * RPT
    ept = E // NS
    ebase = tid * ept
    nb = ept // BEC

    for cl in range(NCH // NC):
        chunk = sc * (NCH // NC) + cl
        _zero_acc_slice(rows, acc, tbase, BEC)
        plsc.subcore_barrier()

        def body(bi, _):
            off = ebase + bi * BEC
            pltpu.sync_copy(src.at[pl.ds(off, BEC)], sidx)
            pltpu.sync_copy(dst.at[pl.ds(off, BEC)], didx)

            def gf(i, _):
                s = sidx[pl.ds(i * 16, 16)]
                gidx[pl.ds(i * 16, 16)] = s * NCH + chunk
                return 0

            lax.fori_loop(0, BEC // 16, gf, 0)
            pltpu.async_copy(h1f.at[gidx], rows, sem).wait()
            pltpu.sync_copy(rows, acc.at[didx], add=True)
            return 0

        lax.fori_loop(0, nb, body, 0)
        plsc.subcore_barrier()
        pltpu.sync_copy(acc.at[pl.ds(tbase, RPT)],
                        out.at[pl.ds(tbase, RPT), pl.ds(chunk * C, C)])
        plsc.subcore_barrier()


def _run_stage_a(x16, src, dst):
    f = pl.kernel(
        _stage_a_body,
        out_type=jax.ShapeDtypeStruct((NC, N, C), jnp.float32),
        mesh=_sc_mesh(),
        compiler_params=pltpu.CompilerParams(use_tc_tiling_on_sc=False),
        scratch_types=[
            pltpu.VMEM((BE,), jnp.int32),
            pltpu.VMEM((BE,), jnp.int32),
            pltpu.VMEM((BE, C), jnp.float32),
            pltpu.VMEM_SHARED((N, C), jnp.float32),
            pltpu.SemaphoreType.DMA,
        ],
    )
    return f(x16, src, dst)


def _run_stage_c(h1f, src, dst):
    f = pl.kernel(
        _stage_c_body,
        out_type=jax.ShapeDtypeStruct((N, D), jnp.float32),
        mesh=_sc_mesh(),
        compiler_params=pltpu.CompilerParams(use_tc_tiling_on_sc=False),
        scratch_types=[
            pltpu.VMEM((BEC,), jnp.int32),
            pltpu.VMEM((BEC,), jnp.int32),
            pltpu.VMEM((BEC,), jnp.int32),
            pltpu.VMEM((BEC, C), jnp.float32),
            pltpu.VMEM_SHARED((N, C), jnp.float32),
            pltpu.SemaphoreType.DMA,
        ],
    )
    return f(h1f, src, dst)


def _h1_body(x_ref, agg_ref, w1l_ref, w1r_ref, b1_ref, out_ref):
    s = agg_ref[0] + agg_ref[1]
    cnt = jnp.maximum(s[:, 2:3], 1.0)
    mean1 = s[:, 0:2] / cnt
    h = jnp.dot(x_ref[...], w1l_ref[...], preferred_element_type=jnp.float32)
    h = h + jnp.dot(mean1, w1r_ref[...], preferred_element_type=jnp.float32)
    out_ref[...] = jnp.maximum(h + b1_ref[...], 0.0)


def _run_h1(x, agg1p, W1l, W1r, b1):
    return pl.pallas_call(
        _h1_body,
        grid=(N // TN,),
        in_specs=[
            pl.BlockSpec((TN, 2), lambda i: (i, 0)),
            pl.BlockSpec((NC, TN, C), lambda i: (0, i, 0)),
            pl.BlockSpec((2, D), lambda i: (0, 0)),
            pl.BlockSpec((2, D), lambda i: (0, 0)),
            pl.BlockSpec((1, D), lambda i: (0, 0)),
        ],
        out_specs=pl.BlockSpec((TN, D), lambda i: (i, 0)),
        out_shape=jax.ShapeDtypeStruct((N, D), jnp.float32),
    )(x, agg1p, W1l, W1r, b1.reshape(1, D))


def _dec_body(h1_ref, agg2_ref, agg1_ref, w2l, w2r, b2, wd1, bdd1, wd2, bdd2,
              wd3, bdd3, out_ref):
    s = agg1_ref[0] + agg1_ref[1]
    cnt = jnp.maximum(s[:, 2:3], 1.0)
    mean2 = agg2_ref[...] / cnt
    h1 = h1_ref[...]
    h2 = jnp.dot(h1, w2l[...], preferred_element_type=jnp.float32)
    h2 = h2 + jnp.dot(mean2, w2r[...], preferred_element_type=jnp.float32)
    h2 = h2 + b2[...]
    z = jnp.maximum(
        jnp.dot(h2, wd1[...], preferred_element_type=jnp.float32) + bdd1[...],
        0.0)
    z = jnp.maximum(
        jnp.dot(z, wd2[...], preferred_element_type=jnp.float32) + bdd2[...],
        0.0)
    out_ref[...] = jnp.dot(z, wd3[...],
                           preferred_element_type=jnp.float32) + bdd3[...]


def _run_decoder(h1, agg2, agg1p, W2l, W2r, b2, Wd1, bd1, Wd2, bd2, Wd3, bd3):
    full = lambda shape: pl.BlockSpec(shape, lambda i: tuple(0 for _ in shape))
    return pl.pallas_call(
        _dec_body,
        grid=(N // TN,),
        in_specs=[
            pl.BlockSpec((TN, D), lambda i: (i, 0)),
            pl.BlockSpec((TN, D), lambda i: (i, 0)),
            pl.BlockSpec((NC, TN, C), lambda i: (0, i, 0)),
            full((D, D)),
            full((D, D)),
            full((1, D)),
            full((D, 128)),
            full((1, 128)),
            full((128, 64)),
            full((1, 64)),
            full((64, 1)),
            full((1, 1)),
        ],
        out_specs=pl.BlockSpec((TN, 1), lambda i: (i, 0)),
        out_shape=jax.ShapeDtypeStruct((N, 1), jnp.float32),
    )(h1, agg2, agg1p, W2l, W2r, b2.reshape(1, D), Wd1, bd1.reshape(1, 128),
      Wd2, bd2.reshape(1, 64), Wd3, bd3.reshape(1, 1))


def kernel(x, edge_index, W1l, W1r, b1, W2l, W2r, b2, Wd1, bd1, Wd2, bd2, Wd3,
           bd3):
    src = edge_index[0].astype(jnp.int32)
    dst = edge_index[1].astype(jnp.int32)
    x16 = jnp.zeros((N, C), jnp.float32).at[:, :2].set(x).at[:, 2].set(1.0)

    agg1p = _run_stage_a(x16, src, dst)
    h1 = _run_h1(x, agg1p, W1l, W1r, b1)
    agg2 = _run_stage_c(h1.reshape(N * NCH, C), src, dst)
    q = _run_decoder(h1, agg2, agg1p, W2l, W2r, b2, Wd1, bd1,
                     Wd2, bd2, Wd3, bd3)
    return q.reshape(N)


# speedup vs baseline: 7.7639x; 1.1056x over previous
import functools

import jax
import jax.numpy as jnp
from jax import lax
from jax.experimental import pallas as pl
from jax.experimental.pallas import tpu as pltpu
from jax.experimental.pallas import tpu_sc as plsc

N = 100000
E = 1600000
D = 128
C = 16
NCH = D // C
NC = 2
NS = 16
BE = 1000
BEC = 800
RPT = N // NS

TN = 2000
IDXB = 3200


def _sc_mesh():
    return plsc.VectorSubcoreMesh(core_axis_name="c", subcore_axis_name="s")


def _zero_acc_slice(rows, acc, tbase, nrows):
    def zf(i, _):
        rows[i, :] = jnp.zeros((C,), jnp.float32)
        return 0

    lax.fori_loop(0, nrows, zf, 0)
    nfull = RPT // nrows
    rem = RPT % nrows
    for k in range(nfull):
        pltpu.sync_copy(rows, acc.at[pl.ds(tbase + k * nrows, nrows)])
    if rem:
        pltpu.sync_copy(rows.at[pl.ds(0, rem)],
                        acc.at[pl.ds(tbase + nfull * nrows, rem)])


def _stage_a_body(x16, src, dst, out, sidx, didx, rows, acc, sem):
    sc = lax.axis_index("c")
    tid = lax.axis_index("s")
    tbase = tid * RPT
    _zero_acc_slice(rows, acc, tbase, BE)
    plsc.subcore_barrier()

    epc = E // NC
    ept = epc // NS
    ebase = sc * epc + tid * ept
    nb = ept // BE

    def body(bi, _):
        off = ebase + bi * BE
        pltpu.sync_copy(src.at[pl.ds(off, BE)], sidx)
        pltpu.sync_copy(dst.at[pl.ds(off, BE)], didx)
        pltpu.async_copy(x16.at[sidx], rows, sem).wait()
        pltpu.sync_copy(rows, acc.at[didx], add=True)
        return 0

    lax.fori_loop(0, nb, body, 0)
    plsc.subcore_barrier()
    pltpu.sync_copy(acc.at[pl.ds(tbase, RPT)], out.at[sc, pl.ds(tbase, RPT)])


def _stage_c_body(h1f, src, dst, out, sidx0, didx0, gidx0, rows0, sidx1,
                  didx1, gidx1, rows1, acc, gsem, ssem0, ssem1):
    sc = lax.axis_index("c")
    tid = lax.axis_index("s")
    tbase = tid * RPT
    ept = E // NS
    ebase = tid * ept
    nb = ept // BEC
    bufs = ((sidx0, didx0, gidx0, rows0, ssem0),
            (sidx1, didx1, gidx1, rows1, ssem1))

    for cl in range(NCH // NC):
        chunk = sc * (NCH // NC) + cl
        _zero_acc_slice(rows0, acc, tbase, BEC)
        plsc.subcore_barrier()

        def run_batch(bi, b, cond_wait):
            sidx, didx, gidx, rows, ssem = bufs[b]
            if cond_wait:
                @pl.when(bi >= 2)
                def _():
                    pltpu.make_async_copy(rows, acc.at[didx], ssem).wait()
            else:
                pltpu.make_async_copy(rows, acc.at[didx], ssem).wait()
            off = ebase + bi * BEC
            pltpu.sync_copy(src.at[pl.ds(off, BEC)], sidx)
            pltpu.sync_copy(dst.at[pl.ds(off, BEC)], didx)

            def gf(i, _):
                s = sidx[pl.ds(i * 16, 16)]
                gidx[pl.ds(i * 16, 16)] = s * NCH + chunk
                return 0

            lax.fori_loop(0, BEC // 16, gf, 0)
            pltpu.async_copy(h1f.at[gidx], rows, gsem).wait()
            pltpu.async_copy(rows, acc.at[didx], ssem, add=True)

        def outer(j, _):
            run_batch(2 * j, 0, True)
            run_batch(2 * j + 1, 1, True)
            return 0

        lax.fori_loop(0, (nb - 1) // 2, outer, 0)
        run_batch(nb - 1, 0, False)
        pltpu.make_async_copy(rows1, acc.at[didx1], ssem1).wait()
        pltpu.make_async_copy(rows0, acc.at[didx0], ssem0).wait()

        plsc.subcore_barrier()
        pltpu.sync_copy(acc.at[pl.ds(tbase, RPT)],
                        out.at[pl.ds(tbase, RPT), pl.ds(chunk * C, C)])
        plsc.subcore_barrier()


def _run_stage_a(x16, src, dst):
    f = pl.kernel(
        _stage_a_body,
        out_type=jax.ShapeDtypeStruct((NC, N, C), jnp.float32),
        mesh=_sc_mesh(),
        compiler_params=pltpu.CompilerParams(use_tc_tiling_on_sc=False),
        scratch_types=[
            pltpu.VMEM((BE,), jnp.int32),
            pltpu.VMEM((BE,), jnp.int32),
            pltpu.VMEM((BE, C), jnp.float32),
            pltpu.VMEM_SHARED((N, C), jnp.float32),
            pltpu.SemaphoreType.DMA,
        ],
    )
    return f(x16, src, dst)


def _run_stage_c(h1f, src, dst):
    f = pl.kernel(
        _stage_c_body,
        out_type=jax.ShapeDtypeStruct((N, D), jnp.float32),
        mesh=_sc_mesh(),
        compiler_params=pltpu.CompilerParams(use_tc_tiling_on_sc=False),
        scratch_types=[
            pltpu.VMEM((BEC,), jnp.int32),
            pltpu.VMEM((BEC,), jnp.int32),
            pltpu.VMEM((BEC,), jnp.int32),
            pltpu.VMEM((BEC, C), jnp.float32),
            pltpu.VMEM((BEC,), jnp.int32),
            pltpu.VMEM((BEC,), jnp.int32),
            pltpu.VMEM((BEC,), jnp.int32),
            pltpu.VMEM((BEC, C), jnp.float32),
            pltpu.VMEM_SHARED((N, C), jnp.float32),
            pltpu.SemaphoreType.DMA,
            pltpu.SemaphoreType.DMA,
            pltpu.SemaphoreType.DMA,
        ],
    )
    return f(h1f, src, dst)


def _h1_body(x_ref, agg_ref, w1l_ref, w1r_ref, b1_ref, out_ref):
    s = agg_ref[0] + agg_ref[1]
    cnt = jnp.maximum(s[:, 2:3], 1.0)
    mean1 = s[:, 0:2] / cnt
    h = jnp.dot(x_ref[...], w1l_ref[...], preferred_element_type=jnp.float32)
    h = h + jnp.dot(mean1, w1r_ref[...], preferred_element_type=jnp.float32)
    out_ref[...] = jnp.maximum(h + b1_ref[...], 0.0)


def _run_h1(x, agg1p, W1l, W1r, b1):
    return pl.pallas_call(
        _h1_body,
        grid=(N // TN,),
        in_specs=[
            pl.BlockSpec((TN, 2), lambda i: (i, 0)),
            pl.BlockSpec((NC, TN, C), lambda i: (0, i, 0)),
            pl.BlockSpec((2, D), lambda i: (0, 0)),
            pl.BlockSpec((2, D), lambda i: (0, 0)),
            pl.BlockSpec((1, D), lambda i: (0, 0)),
        ],
        out_specs=pl.BlockSpec((TN, D), lambda i: (i, 0)),
        out_shape=jax.ShapeDtypeStruct((N, D), jnp.float32),
    )(x, agg1p, W1l, W1r, b1.reshape(1, D))


def _dec_body(h1_ref, agg2_ref, agg1_ref, w2l, w2r, b2, wd1, bdd1, wd2, bdd2,
              wd3, bdd3, out_ref):
    s = agg1_ref[0] + agg1_ref[1]
    cnt = jnp.maximum(s[:, 2:3], 1.0)
    mean2 = agg2_ref[...] / cnt
    h1 = h1_ref[...]
    h2 = jnp.dot(h1, w2l[...], preferred_element_type=jnp.float32)
    h2 = h2 + jnp.dot(mean2, w2r[...], preferred_element_type=jnp.float32)
    h2 = h2 + b2[...]
    z = jnp.maximum(
        jnp.dot(h2, wd1[...], preferred_element_type=jnp.float32) + bdd1[...],
        0.0)
    z = jnp.maximum(
        jnp.dot(z, wd2[...], preferred_element_type=jnp.float32) + bdd2[...],
        0.0)
    out_ref[...] = jnp.dot(z, wd3[...],
                           preferred_element_type=jnp.float32) + bdd3[...]


def _run_decoder(h1, agg2, agg1p, W2l, W2r, b2, Wd1, bd1, Wd2, bd2, Wd3, bd3):
    full = lambda shape: pl.BlockSpec(shape, lambda i: tuple(0 for _ in shape))
    return pl.pallas_call(
        _dec_body,
        grid=(N // TN,),
        in_specs=[
            pl.BlockSpec((TN, D), lambda i: (i, 0)),
            pl.BlockSpec((TN, D), lambda i: (i, 0)),
            pl.BlockSpec((NC, TN, C), lambda i: (0, i, 0)),
            full((D, D)),
            full((D, D)),
            full((1, D)),
            full((D, 128)),
            full((1, 128)),
            full((128, 64)),
            full((1, 64)),
            full((64, 1)),
            full((1, 1)),
        ],
        out_specs=pl.BlockSpec((TN, 1), lambda i: (i, 0)),
        out_shape=jax.ShapeDtypeStruct((N, 1), jnp.float32),
    )(h1, agg2, agg1p, W2l, W2r, b2.reshape(1, D), Wd1, bd1.reshape(1, 128),
      Wd2, bd2.reshape(1, 64), Wd3, bd3.reshape(1, 1))


def kernel(x, edge_index, W1l, W1r, b1, W2l, W2r, b2, Wd1, bd1, Wd2, bd2, Wd3,
           bd3):
    src = edge_index[0].astype(jnp.int32)
    dst = edge_index[1].astype(jnp.int32)
    x16 = jnp.zeros((N, C), jnp.float32).at[:, :2].set(x).at[:, 2].set(1.0)

    agg1p = _run_stage_a(x16, src, dst)
    h1 = _run_h1(x, agg1p, W1l, W1r, b1)
    agg2 = _run_stage_c(h1.reshape(N * NCH, C), src, dst)
    q = _run_decoder(h1, agg2, agg1p, W2l, W2r, b2, Wd1, bd1,
                     Wd2, bd2, Wd3, bd3)
    return q.reshape(N)


# speedup vs baseline: 8.5245x; 1.0980x over previous
import functools

import jax
import jax.numpy as jnp
from jax import lax
from jax.experimental import pallas as pl
from jax.experimental.pallas import tpu as pltpu
from jax.experimental.pallas import tpu_sc as plsc

N = 100000
E = 1600000
D = 128
C = 16
NCH = D // C
NC = 2
NS = 16
BE = 1000
BEC = 800
RPT = N // NS

TN = 2000
IDXB = 3200


def _sc_mesh():
    return plsc.VectorSubcoreMesh(core_axis_name="c", subcore_axis_name="s")


def _zero_acc_slice(rows, acc, tbase, nrows):
    def zf(i, _):
        rows[i, :] = jnp.zeros((C,), jnp.float32)
        return 0

    lax.fori_loop(0, nrows, zf, 0)
    nfull = RPT // nrows
    rem = RPT % nrows
    for k in range(nfull):
        pltpu.sync_copy(rows, acc.at[pl.ds(tbase + k * nrows, nrows)])
    if rem:
        pltpu.sync_copy(rows.at[pl.ds(0, rem)],
                        acc.at[pl.ds(tbase + nfull * nrows, rem)])


def _stage_a_body(x16, ei, out, sidx, didx, rows, acc, sem):
    sc = lax.axis_index("c")
    tid = lax.axis_index("s")
    tbase = tid * RPT
    _zero_acc_slice(rows, acc, tbase, BE)
    plsc.subcore_barrier()

    epc = E // NC
    ept = epc // NS
    ebase = sc * epc + tid * ept
    nb = ept // BE

    def body(bi, _):
        off = ebase + bi * BE
        pltpu.sync_copy(ei.at[0, pl.ds(off, BE)], sidx)
        pltpu.sync_copy(ei.at[1, pl.ds(off, BE)], didx)
        pltpu.async_copy(x16.at[sidx], rows, sem).wait()
        pltpu.sync_copy(rows, acc.at[didx], add=True)
        return 0

    lax.fori_loop(0, nb, body, 0)
    plsc.subcore_barrier()
    pltpu.sync_copy(acc.at[pl.ds(tbase, RPT)], out.at[sc, pl.ds(tbase, RPT)])


def _stage_c_body(h1f, ei, out, sidx0, didx0, gidx0, rows0, sidx1,
                  didx1, gidx1, rows1, acc, gsem, ssem0, ssem1):
    sc = lax.axis_index("c")
    tid = lax.axis_index("s")
    tbase = tid * RPT
    ept = E // NS
    ebase = tid * ept
    nb = ept // BEC
    bufs = ((sidx0, didx0, gidx0, rows0, ssem0),
            (sidx1, didx1, gidx1, rows1, ssem1))

    for cl in range(NCH // NC):
        chunk = sc * (NCH // NC) + cl
        _zero_acc_slice(rows0, acc, tbase, BEC)
        plsc.subcore_barrier()

        def run_batch(bi, b, cond_wait):
            sidx, didx, gidx, rows, ssem = bufs[b]
            if cond_wait:
                @pl.when(bi >= 2)
                def _():
                    pltpu.make_async_copy(rows, acc.at[didx], ssem).wait()
            else:
                pltpu.make_async_copy(rows, acc.at[didx], ssem).wait()
            off = ebase + bi * BEC
            pltpu.sync_copy(ei.at[0, pl.ds(off, BEC)], sidx)
            pltpu.sync_copy(ei.at[1, pl.ds(off, BEC)], didx)

            def gf(i, _):
                s = sidx[pl.ds(i * 16, 16)]
                gidx[pl.ds(i * 16, 16)] = s * NCH + chunk
                return 0

            lax.fori_loop(0, BEC // 16, gf, 0)
            pltpu.async_copy(h1f.at[gidx], rows, gsem).wait()
            pltpu.async_copy(rows, acc.at[didx], ssem, add=True)

        def outer(j, _):
            run_batch(2 * j, 0, True)
            run_batch(2 * j + 1, 1, True)
            return 0

        lax.fori_loop(0, (nb - 1) // 2, outer, 0)
        run_batch(nb - 1, 0, False)
        pltpu.make_async_copy(rows1, acc.at[didx1], ssem1).wait()
        pltpu.make_async_copy(rows0, acc.at[didx0], ssem0).wait()

        plsc.subcore_barrier()
        pltpu.sync_copy(acc.at[pl.ds(tbase, RPT)],
                        out.at[pl.ds(tbase, RPT), pl.ds(chunk * C, C)])
        plsc.subcore_barrier()


def _run_stage_a(x16, ei):
    f = pl.kernel(
        _stage_a_body,
        out_type=jax.ShapeDtypeStruct((NC, N, C), jnp.float32),
        mesh=_sc_mesh(),
        compiler_params=pltpu.CompilerParams(use_tc_tiling_on_sc=False),
        scratch_types=[
            pltpu.VMEM((BE,), jnp.int32),
            pltpu.VMEM((BE,), jnp.int32),
            pltpu.VMEM((BE, C), jnp.float32),
            pltpu.VMEM_SHARED((N, C), jnp.float32),
            pltpu.SemaphoreType.DMA,
        ],
    )
    return f(x16, ei)


def _run_stage_c(h1f, ei):
    f = pl.kernel(
        _stage_c_body,
        out_type=jax.ShapeDtypeStruct((N, D), jnp.float32),
        mesh=_sc_mesh(),
        compiler_params=pltpu.CompilerParams(use_tc_tiling_on_sc=False),
        scratch_types=[
            pltpu.VMEM((BEC,), jnp.int32),
            pltpu.VMEM((BEC,), jnp.int32),
            pltpu.VMEM((BEC,), jnp.int32),
            pltpu.VMEM((BEC, C), jnp.float32),
            pltpu.VMEM((BEC,), jnp.int32),
            pltpu.VMEM((BEC,), jnp.int32),
            pltpu.VMEM((BEC,), jnp.int32),
            pltpu.VMEM((BEC, C), jnp.float32),
            pltpu.VMEM_SHARED((N, C), jnp.float32),
            pltpu.SemaphoreType.DMA,
            pltpu.SemaphoreType.DMA,
            pltpu.SemaphoreType.DMA,
        ],
    )
    return f(h1f, ei)


def _x16_body(x_ref, out_ref):
    blk = jnp.concatenate(
        [x_ref[...],
         jnp.ones((TN, 1), jnp.float32),
         jnp.zeros((TN, C - 3), jnp.float32)], axis=1)
    out_ref[...] = blk


def _build_x16(x):
    return pl.pallas_call(
        _x16_body,
        grid=(N // TN,),
        in_specs=[pl.BlockSpec((TN, 2), lambda i: (i, 0))],
        out_specs=pl.BlockSpec((TN, C), lambda i: (i, 0)),
        out_shape=jax.ShapeDtypeStruct((N, C), jnp.float32),
    )(x)


def _h1_body(x_ref, agg_ref, w1l_ref, w1r_ref, b1_ref, out_ref):
    s = agg_ref[0] + agg_ref[1]
    cnt = jnp.maximum(s[:, 2:3], 1.0)
    mean1 = s[:, 0:2] / cnt
    h = jnp.dot(x_ref[...], w1l_ref[...], preferred_element_type=jnp.float32)
    h = h + jnp.dot(mean1, w1r_ref[...], preferred_element_type=jnp.float32)
    out_ref[...] = jnp.maximum(h + b1_ref[...], 0.0)


def _run_h1(x, agg1p, W1l, W1r, b1):
    return pl.pallas_call(
        _h1_body,
        grid=(N // TN,),
        in_specs=[
            pl.BlockSpec((TN, 2), lambda i: (i, 0)),
            pl.BlockSpec((NC, TN, C), lambda i: (0, i, 0)),
            pl.BlockSpec((2, D), lambda i: (0, 0)),
            pl.BlockSpec((2, D), lambda i: (0, 0)),
            pl.BlockSpec((1, D), lambda i: (0, 0)),
        ],
        out_specs=pl.BlockSpec((TN, D), lambda i: (i, 0)),
        out_shape=jax.ShapeDtypeStruct((N, D), jnp.float32),
    )(x, agg1p, W1l, W1r, b1.reshape(1, D))


def _dec_body(h1_ref, agg2_ref, agg1_ref, w2l, w2r, b2, wd1, bdd1, wd2, bdd2,
              wd3, bdd3, out_ref):
    s = agg1_ref[0] + agg1_ref[1]
    cnt = jnp.maximum(s[:, 2:3], 1.0)
    mean2 = agg2_ref[...] / cnt
    h1 = h1_ref[...]
    h2 = jnp.dot(h1, w2l[...], preferred_element_type=jnp.float32)
    h2 = h2 + jnp.dot(mean2, w2r[...], preferred_element_type=jnp.float32)
    h2 = h2 + b2[...]
    z = jnp.maximum(
        jnp.dot(h2, wd1[...], preferred_element_type=jnp.float32) + bdd1[...],
        0.0)
    z = jnp.maximum(
        jnp.dot(z, wd2[...], preferred_element_type=jnp.float32) + bdd2[...],
        0.0)
    out_ref[...] = jnp.dot(z, wd3[...],
                           preferred_element_type=jnp.float32) + bdd3[...]


def _run_decoder(h1, agg2, agg1p, W2l, W2r, b2, Wd1, bd1, Wd2, bd2, Wd3, bd3):
    full = lambda shape: pl.BlockSpec(shape, lambda i: tuple(0 for _ in shape))
    return pl.pallas_call(
        _dec_body,
        grid=(N // TN,),
        in_specs=[
            pl.BlockSpec((TN, D), lambda i: (i, 0)),
            pl.BlockSpec((TN, D), lambda i: (i, 0)),
            pl.BlockSpec((NC, TN, C), lambda i: (0, i, 0)),
            full((D, D)),
            full((D, D)),
            full((1, D)),
            full((D, 128)),
            full((1, 128)),
            full((128, 64)),
            full((1, 64)),
            full((64, 1)),
            full((1, 1)),
        ],
        out_specs=pl.BlockSpec((TN, 1), lambda i: (i, 0)),
        out_shape=jax.ShapeDtypeStruct((N, 1), jnp.float32),
    )(h1, agg2, agg1p, W2l, W2r, b2.reshape(1, D), Wd1, bd1.reshape(1, 128),
      Wd2, bd2.reshape(1, 64), Wd3, bd3.reshape(1, 1))


def kernel(x, edge_index, W1l, W1r, b1, W2l, W2r, b2, Wd1, bd1, Wd2, bd2, Wd3,
           bd3):
    ei = edge_index.astype(jnp.int32)
    x16 = _build_x16(x)

    agg1p = _run_stage_a(x16, ei)
    h1 = _run_h1(x, agg1p, W1l, W1r, b1)
    agg2 = _run_stage_c(h1.reshape(N * NCH, C), ei)
    q = _run_decoder(h1, agg2, agg1p, W2l, W2r, b2, Wd1, bd1,
                     Wd2, bd2, Wd3, bd3)
    return q.reshape(N)


# speedup vs baseline: 10.9654x; 1.2863x over previous
import functools

import jax
import jax.numpy as jnp
from jax import lax
from jax.experimental import pallas as pl
from jax.experimental.pallas import tpu as pltpu
from jax.experimental.pallas import tpu_sc as plsc

N = 100000
E = 1600000
D = 128
C = 16
NCH = D // C
NC = 2
NS = 16
BE = 1000
BEC = 800
RPT = N // NS

TN = 2000
IDXB = 3200


def _sc_mesh():
    return plsc.VectorSubcoreMesh(core_axis_name="c", subcore_axis_name="s")


def _zero_acc_slice(rows, acc, tbase, nrows):
    def zf(i, _):
        rows[i, :] = jnp.zeros((C,), jnp.float32)
        return 0

    lax.fori_loop(0, nrows, zf, 0)
    nfull = RPT // nrows
    rem = RPT % nrows
    for k in range(nfull):
        pltpu.sync_copy(rows, acc.at[pl.ds(tbase + k * nrows, nrows)])
    if rem:
        pltpu.sync_copy(rows.at[pl.ds(0, rem)],
                        acc.at[pl.ds(tbase + nfull * nrows, rem)])


def _stage_a_body(x16, ei, out, sidx, didx, rows, acc, sem):
    sc = lax.axis_index("c")
    tid = lax.axis_index("s")
    tbase = tid * RPT
    _zero_acc_slice(rows, acc, tbase, BE)
    plsc.subcore_barrier()

    epc = E // NC
    ept = epc // NS
    ebase = sc * epc + tid * ept
    nb = ept // BE

    def body(bi, _):
        off = ebase + bi * BE
        pltpu.sync_copy(ei.at[0, pl.ds(off, BE)], sidx)
        pltpu.sync_copy(ei.at[1, pl.ds(off, BE)], didx)
        pltpu.async_copy(x16.at[sidx], rows, sem).wait()
        pltpu.sync_copy(rows, acc.at[didx], add=True)
        return 0

    lax.fori_loop(0, nb, body, 0)
    plsc.subcore_barrier()
    pltpu.sync_copy(acc.at[pl.ds(tbase, RPT)], out.at[sc, pl.ds(tbase, RPT)])


def _stage_c_body(h1f, ei, out, gidx0, gidx1, didx0, didx1, didx2, didx3,
                  rows0, rows1, acc, isem, gsem, ssem0, ssem1):
    sc = lax.axis_index("c")
    tid = lax.axis_index("s")
    tbase = tid * RPT
    ept = E // NS
    ebase = tid * ept
    nb = ept // BEC
    gidx = (gidx0, gidx1)
    didx = (didx0, didx1, didx2, didx3)
    rows = (rows0, rows1)
    ssem = (ssem0, ssem1)

    for cl in range(NCH // NC):
        chunk = sc * (NCH // NC) + cl
        _zero_acc_slice(rows0, acc, tbase, BEC)
        plsc.subcore_barrier()

        def run_batch(bi, k):
            b, c = k % 2, k % 4
            gx, dx, rw = gidx[b], didx[c], rows[b]
            off = ebase + bi * BEC
            pltpu.make_async_copy(ei.at[0, pl.ds(off, BEC)], gx, isem).wait()
            pltpu.make_async_copy(ei.at[1, pl.ds(off, BEC)], dx, isem).wait()

            def gf(t, _):
                v = gx[pl.ds(t * 16, 16)]
                gx[pl.ds(t * 16, 16)] = v * NCH + chunk
                return 0

            lax.fori_loop(0, BEC // 16, gf, 0)

            @pl.when(bi >= 2)
            def _():
                pltpu.make_async_copy(rw, acc.at[dx], ssem[b]).wait()

            gd = pltpu.async_copy(h1f.at[gx], rw, gsem)

            @pl.when(bi < nb - 1)
            def _():
                off2 = off + BEC
                pltpu.async_copy(ei.at[0, pl.ds(off2, BEC)],
                                 gidx[(b + 1) % 2], isem)
                pltpu.async_copy(ei.at[1, pl.ds(off2, BEC)],
                                 didx[(c + 1) % 4], isem)

            gd.wait()
            pltpu.async_copy(rw, acc.at[dx], ssem[b], add=True)

        pltpu.async_copy(ei.at[0, pl.ds(ebase, BEC)], gidx0, isem)
        pltpu.async_copy(ei.at[1, pl.ds(ebase, BEC)], didx0, isem)

        def outer(j, _):
            for k in range(4):
                run_batch(4 * j + k, k)
            return 0

        lax.fori_loop(0, (nb - 1) // 4, outer, 0)
        run_batch(nb - 1, 0)
        pltpu.make_async_copy(rows1, acc.at[didx3], ssem1).wait()
        pltpu.make_async_copy(rows0, acc.at[didx0], ssem0).wait()

        plsc.subcore_barrier()
        pltpu.sync_copy(acc.at[pl.ds(tbase, RPT)],
                        out.at[pl.ds(tbase, RPT), pl.ds(chunk * C, C)])
        plsc.subcore_barrier()


def _run_stage_a(x16, ei):
    f = pl.kernel(
        _stage_a_body,
        out_type=jax.ShapeDtypeStruct((NC, N, C), jnp.float32),
        mesh=_sc_mesh(),
        compiler_params=pltpu.CompilerParams(use_tc_tiling_on_sc=False),
        scratch_types=[
            pltpu.VMEM((BE,), jnp.int32),
            pltpu.VMEM((BE,), jnp.int32),
            pltpu.VMEM((BE, C), jnp.float32),
            pltpu.VMEM_SHARED((N, C), jnp.float32),
            pltpu.SemaphoreType.DMA,
        ],
    )
    return f(x16, ei)


def _run_stage_c(h1f, ei):
    f = pl.kernel(
        _stage_c_body,
        out_type=jax.ShapeDtypeStruct((N, D), jnp.float32),
        mesh=_sc_mesh(),
        compiler_params=pltpu.CompilerParams(use_tc_tiling_on_sc=False),
        scratch_types=[
            pltpu.VMEM((BEC,), jnp.int32),
            pltpu.VMEM((BEC,), jnp.int32),
            pltpu.VMEM((BEC,), jnp.int32),
            pltpu.VMEM((BEC,), jnp.int32),
            pltpu.VMEM((BEC,), jnp.int32),
            pltpu.VMEM((BEC,), jnp.int32),
            pltpu.VMEM((BEC, C), jnp.float32),
            pltpu.VMEM((BEC, C), jnp.float32),
            pltpu.VMEM_SHARED((N, C), jnp.float32),
            pltpu.SemaphoreType.DMA,
            pltpu.SemaphoreType.DMA,
            pltpu.SemaphoreType.DMA,
            pltpu.SemaphoreType.DMA,
        ],
    )
    return f(h1f, ei)


def _x16_body(x_ref, out_ref):
    blk = jnp.concatenate(
        [x_ref[...],
         jnp.ones((TN, 1), jnp.float32),
         jnp.zeros((TN, C - 3), jnp.float32)], axis=1)
    out_ref[...] = blk


def _build_x16(x):
    return pl.pallas_call(
        _x16_body,
        grid=(N // TN,),
        in_specs=[pl.BlockSpec((TN, 2), lambda i: (i, 0))],
        out_specs=pl.BlockSpec((TN, C), lambda i: (i, 0)),
        out_shape=jax.ShapeDtypeStruct((N, C), jnp.float32),
    )(x)


def _h1_body(x_ref, agg_ref, w1l_ref, w1r_ref, b1_ref, out_ref):
    s = agg_ref[0] + agg_ref[1]
    cnt = jnp.maximum(s[:, 2:3], 1.0)
    mean1 = s[:, 0:2] / cnt
    h = jnp.dot(x_ref[...], w1l_ref[...], preferred_element_type=jnp.float32)
    h = h + jnp.dot(mean1, w1r_ref[...], preferred_element_type=jnp.float32)
    out_ref[...] = jnp.maximum(h + b1_ref[...], 0.0)


def _run_h1(x, agg1p, W1l, W1r, b1):
    return pl.pallas_call(
        _h1_body,
        grid=(N // TN,),
        in_specs=[
            pl.BlockSpec((TN, 2), lambda i: (i, 0)),
            pl.BlockSpec((NC, TN, C), lambda i: (0, i, 0)),
            pl.BlockSpec((2, D), lambda i: (0, 0)),
            pl.BlockSpec((2, D), lambda i: (0, 0)),
            pl.BlockSpec((1, D), lambda i: (0, 0)),
        ],
        out_specs=pl.BlockSpec((TN, D), lambda i: (i, 0)),
        out_shape=jax.ShapeDtypeStruct((N, D), jnp.float32),
    )(x, agg1p, W1l, W1r, b1.reshape(1, D))


def _dec_body(h1_ref, agg2_ref, agg1_ref, w2l, w2r, b2, wd1, bdd1, wd2, bdd2,
              wd3, bdd3, out_ref):
    s = agg1_ref[0] + agg1_ref[1]
    cnt = jnp.maximum(s[:, 2:3], 1.0)
    mean2 = agg2_ref[...] / cnt
    h1 = h1_ref[...]
    h2 = jnp.dot(h1, w2l[...], preferred_element_type=jnp.float32)
    h2 = h2 + jnp.dot(mean2, w2r[...], preferred_element_type=jnp.float32)
    h2 = h2 + b2[...]
    z = jnp.maximum(
        jnp.dot(h2, wd1[...], preferred_element_type=jnp.float32) + bdd1[...],
        0.0)
    z = jnp.maximum(
        jnp.dot(z, wd2[...], preferred_element_type=jnp.float32) + bdd2[...],
        0.0)
    out_ref[...] = jnp.dot(z, wd3[...],
                           preferred_element_type=jnp.float32) + bdd3[...]


def _run_decoder(h1, agg2, agg1p, W2l, W2r, b2, Wd1, bd1, Wd2, bd2, Wd3, bd3):
    full = lambda shape: pl.BlockSpec(shape, lambda i: tuple(0 for _ in shape))
    return pl.pallas_call(
        _dec_body,
        grid=(N // TN,),
        in_specs=[
            pl.BlockSpec((TN, D), lambda i: (i, 0)),
            pl.BlockSpec((TN, D), lambda i: (i, 0)),
            pl.BlockSpec((NC, TN, C), lambda i: (0, i, 0)),
            full((D, D)),
            full((D, D)),
            full((1, D)),
            full((D, 128)),
            full((1, 128)),
            full((128, 64)),
            full((1, 64)),
            full((64, 1)),
            full((1, 1)),
        ],
        out_specs=pl.BlockSpec((TN, 1), lambda i: (i, 0)),
        out_shape=jax.ShapeDtypeStruct((N, 1), jnp.float32),
    )(h1, agg2, agg1p, W2l, W2r, b2.reshape(1, D), Wd1, bd1.reshape(1, 128),
      Wd2, bd2.reshape(1, 64), Wd3, bd3.reshape(1, 1))


def kernel(x, edge_index, W1l, W1r, b1, W2l, W2r, b2, Wd1, bd1, Wd2, bd2, Wd3,
           bd3):
    ei = edge_index.astype(jnp.int32)
    x16 = _build_x16(x)

    agg1p = _run_stage_a(x16, ei)
    h1 = _run_h1(x, agg1p, W1l, W1r, b1)
    agg2 = _run_stage_c(h1.reshape(N * NCH, C), ei)
    q = _run_decoder(h1, agg2, agg1p, W2l, W2r, b2, Wd1, bd1,
                     Wd2, bd2, Wd3, bd3)
    return q.reshape(N)


# speedup vs baseline: 11.0525x; 1.0079x over previous
import functools

import jax
import jax.numpy as jnp
from jax import lax
from jax.experimental import pallas as pl
from jax.experimental.pallas import tpu as pltpu
from jax.experimental.pallas import tpu_sc as plsc

N = 100000
E = 1600000
D = 128
C = 16
NCH = D // C
NC = 2
NS = 16
BE = 1000
BEC = 800
RPT = N // NS

TN = 2000
IDXB = 3200


def _sc_mesh():
    return plsc.VectorSubcoreMesh(core_axis_name="c", subcore_axis_name="s")


def _zero_acc_slice(rows, acc, tbase, nrows):
    def zf(i, _):
        rows[i, :] = jnp.zeros((C,), jnp.float32)
        return 0

    lax.fori_loop(0, nrows, zf, 0)
    nfull = RPT // nrows
    rem = RPT % nrows
    for k in range(nfull):
        pltpu.sync_copy(rows, acc.at[pl.ds(tbase + k * nrows, nrows)])
    if rem:
        pltpu.sync_copy(rows.at[pl.ds(0, rem)],
                        acc.at[pl.ds(tbase + nfull * nrows, rem)])


def _stage_a_body(x16, ei, out, sidx, didx, rows, acc, sem):
    sc = lax.axis_index("c")
    tid = lax.axis_index("s")
    tbase = tid * RPT
    _zero_acc_slice(rows, acc, tbase, BE)
    plsc.subcore_barrier()

    epc = E // NC
    ept = epc // NS
    ebase = sc * epc + tid * ept
    nb = ept // BE

    def body(bi, _):
        off = ebase + bi * BE
        pltpu.sync_copy(ei.at[pl.ds(off, BE)], sidx)
        pltpu.sync_copy(ei.at[pl.ds(E + off, BE)], didx)
        pltpu.async_copy(x16.at[sidx], rows, sem).wait()
        pltpu.sync_copy(rows, acc.at[didx], add=True)
        return 0

    lax.fori_loop(0, nb, body, 0)
    plsc.subcore_barrier()
    pltpu.sync_copy(acc.at[pl.ds(tbase, RPT)], out.at[sc, pl.ds(tbase, RPT)])


def _stage_c_body(h1f, ei, out, gidx0, gidx1, didx0, didx1, didx2, didx3,
                  rows0, rows1, acc, isem, gsem, ssem0, ssem1):
    sc = lax.axis_index("c")
    tid = lax.axis_index("s")
    tbase = tid * RPT
    ept = E // NS
    ebase = tid * ept
    nb = ept // BEC
    gidx = (gidx0, gidx1)
    didx = (didx0, didx1, didx2, didx3)
    rows = (rows0, rows1)
    ssem = (ssem0, ssem1)

    for cl in range(NCH // NC):
        chunk = sc * (NCH // NC) + cl
        _zero_acc_slice(rows0, acc, tbase, BEC)
        plsc.subcore_barrier()

        def run_batch(bi, k):
            b, c = k % 2, k % 4
            gx, dx, rw = gidx[b], didx[c], rows[b]
            off = ebase + bi * BEC
            pltpu.make_async_copy(ei.at[pl.ds(off, BEC)], gx, isem).wait()
            pltpu.make_async_copy(ei.at[pl.ds(E + off, BEC)], dx, isem).wait()

            def gf(t, _):
                v = gx[pl.ds(t * 16, 16)]
                gx[pl.ds(t * 16, 16)] = v * NCH + chunk
                return 0

            lax.fori_loop(0, BEC // 16, gf, 0)

            @pl.when(bi >= 2)
            def _():
                pltpu.make_async_copy(rw, acc.at[dx], ssem[b]).wait()

            gd = pltpu.async_copy(h1f.at[gx], rw, gsem)

            @pl.when(bi < nb - 1)
            def _():
                off2 = off + BEC
                pltpu.async_copy(ei.at[pl.ds(off2, BEC)],
                                 gidx[(b + 1) % 2], isem)
                pltpu.async_copy(ei.at[pl.ds(E + off2, BEC)],
                                 didx[(c + 1) % 4], isem)

            gd.wait()
            pltpu.async_copy(rw, acc.at[dx], ssem[b], add=True)

        pltpu.async_copy(ei.at[pl.ds(ebase, BEC)], gidx0, isem)
        pltpu.async_copy(ei.at[pl.ds(E + ebase, BEC)], didx0, isem)

        def outer(j, _):
            for k in range(4):
                run_batch(4 * j + k, k)
            return 0

        lax.fori_loop(0, (nb - 1) // 4, outer, 0)
        run_batch(nb - 1, 0)
        pltpu.make_async_copy(rows1, acc.at[didx3], ssem1).wait()
        pltpu.make_async_copy(rows0, acc.at[didx0], ssem0).wait()

        plsc.subcore_barrier()
        pltpu.sync_copy(acc.at[pl.ds(tbase, RPT)],
                        out.at[pl.ds(tbase, RPT), pl.ds(chunk * C, C)])
        plsc.subcore_barrier()


def _run_stage_a(x16, ei):
    f = pl.kernel(
        _stage_a_body,
        out_type=jax.ShapeDtypeStruct((NC, N, C), jnp.float32),
        mesh=_sc_mesh(),
        compiler_params=pltpu.CompilerParams(use_tc_tiling_on_sc=False),
        scratch_types=[
            pltpu.VMEM((BE,), jnp.int32),
            pltpu.VMEM((BE,), jnp.int32),
            pltpu.VMEM((BE, C), jnp.float32),
            pltpu.VMEM_SHARED((N, C), jnp.float32),
            pltpu.SemaphoreType.DMA,
        ],
    )
    return f(x16, ei)


def _run_stage_c(h1f, ei):
    f = pl.kernel(
        _stage_c_body,
        out_type=jax.ShapeDtypeStruct((N, D), jnp.float32),
        mesh=_sc_mesh(),
        compiler_params=pltpu.CompilerParams(use_tc_tiling_on_sc=False),
        scratch_types=[
            pltpu.VMEM((BEC,), jnp.int32),
            pltpu.VMEM((BEC,), jnp.int32),
            pltpu.VMEM((BEC,), jnp.int32),
            pltpu.VMEM((BEC,), jnp.int32),
            pltpu.VMEM((BEC,), jnp.int32),
            pltpu.VMEM((BEC,), jnp.int32),
            pltpu.VMEM((BEC, C), jnp.float32),
            pltpu.VMEM((BEC, C), jnp.float32),
            pltpu.VMEM_SHARED((N, C), jnp.float32),
            pltpu.SemaphoreType.DMA,
            pltpu.SemaphoreType.DMA,
            pltpu.SemaphoreType.DMA,
            pltpu.SemaphoreType.DMA,
        ],
    )
    return f(h1f, ei)


TNX = 4000


def _x16_body(x_ref, out_ref):
    blk = jnp.concatenate(
        [x_ref[...],
         jnp.ones((TNX, 1), jnp.float32),
         jnp.zeros((TNX, C - 3), jnp.float32)], axis=1)
    out_ref[...] = blk


def _build_x16(x):
    return pl.pallas_call(
        _x16_body,
        grid=(N // TNX,),
        in_specs=[pl.BlockSpec((TNX, 2), lambda i: (i, 0))],
        out_specs=pl.BlockSpec((TNX, C), lambda i: (i, 0)),
        out_shape=jax.ShapeDtypeStruct((N, C), jnp.float32),
    )(x)


def _h1_body(x_ref, agg_ref, w1l_ref, w1r_ref, b1_ref, out_ref):
    s = agg_ref[0] + agg_ref[1]
    cnt = jnp.maximum(s[:, 2:3], 1.0)
    mean1 = s[:, 0:2] / cnt
    h = jnp.dot(x_ref[...], w1l_ref[...], preferred_element_type=jnp.float32)
    h = h + jnp.dot(mean1, w1r_ref[...], preferred_element_type=jnp.float32)
    out_ref[...] = jnp.maximum(h + b1_ref[...], 0.0)


def _run_h1(x, agg1p, W1l, W1r, b1):
    return pl.pallas_call(
        _h1_body,
        grid=(N // TN,),
        in_specs=[
            pl.BlockSpec((TN, 2), lambda i: (i, 0)),
            pl.BlockSpec((NC, TN, C), lambda i: (0, i, 0)),
            pl.BlockSpec((2, D), lambda i: (0, 0)),
            pl.BlockSpec((2, D), lambda i: (0, 0)),
            pl.BlockSpec((1, D), lambda i: (0, 0)),
        ],
        out_specs=pl.BlockSpec((TN, D), lambda i: (i, 0)),
        out_shape=jax.ShapeDtypeStruct((N, D), jnp.float32),
    )(x, agg1p, W1l, W1r, b1.reshape(1, D))


def _dec_body(h1_ref, agg2_ref, agg1_ref, w2l, w2r, b2, wd1, bdd1, wd2, bdd2,
              wd3, bdd3, out_ref):
    s = agg1_ref[0] + agg1_ref[1]
    cnt = jnp.maximum(s[:, 2:3], 1.0)
    mean2 = agg2_ref[...] / cnt
    h1 = h1_ref[...]
    h2 = jnp.dot(h1, w2l[...], preferred_element_type=jnp.float32)
    h2 = h2 + jnp.dot(mean2, w2r[...], preferred_element_type=jnp.float32)
    h2 = h2 + b2[...]
    z = jnp.maximum(
        jnp.dot(h2, wd1[...], preferred_element_type=jnp.float32) + bdd1[...],
        0.0)
    z = jnp.maximum(
        jnp.dot(z, wd2[...], preferred_element_type=jnp.float32) + bdd2[...],
        0.0)
    out_ref[...] = jnp.dot(z, wd3[...],
                           preferred_element_type=jnp.float32) + bdd3[...]


def _run_decoder(h1, agg2, agg1p, W2l, W2r, b2, Wd1, bd1, Wd2, bd2, Wd3, bd3):
    full = lambda shape: pl.BlockSpec(shape, lambda i: tuple(0 for _ in shape))
    return pl.pallas_call(
        _dec_body,
        grid=(N // TN,),
        in_specs=[
            pl.BlockSpec((TN, D), lambda i: (i, 0)),
            pl.BlockSpec((TN, D), lambda i: (i, 0)),
            pl.BlockSpec((NC, TN, C), lambda i: (0, i, 0)),
            full((D, D)),
            full((D, D)),
            full((1, D)),
            full((D, 128)),
            full((1, 128)),
            full((128, 64)),
            full((1, 64)),
            full((64, 1)),
            full((1, 1)),
        ],
        out_specs=pl.BlockSpec((TN, 1), lambda i: (i, 0)),
        out_shape=jax.ShapeDtypeStruct((N, 1), jnp.float32),
    )(h1, agg2, agg1p, W2l, W2r, b2.reshape(1, D), Wd1, bd1.reshape(1, 128),
      Wd2, bd2.reshape(1, 64), Wd3, bd3.reshape(1, 1))


def kernel(x, edge_index, W1l, W1r, b1, W2l, W2r, b2, Wd1, bd1, Wd2, bd2, Wd3,
           bd3):
    ei = edge_index.astype(jnp.int32).reshape(2 * E)
    x16 = _build_x16(x)

    agg1p = _run_stage_a(x16, ei)
    h1 = _run_h1(x, agg1p, W1l, W1r, b1)
    agg2 = _run_stage_c(h1.reshape(N * NCH, C), ei)
    q = _run_decoder(h1, agg2, agg1p, W2l, W2r, b2, Wd1, bd1,
                     Wd2, bd2, Wd3, bd3)
    return q.reshape(N)


# speedup vs baseline: 11.5875x; 1.0484x over previous
import functools

import jax
import jax.numpy as jnp
from jax import lax
from jax.experimental import pallas as pl
from jax.experimental.pallas import tpu as pltpu
from jax.experimental.pallas import tpu_sc as plsc

N = 100000
E = 1600000
D = 128
C = 16
NCH = D // C
NC = 2
NS = 16
BEA = 800
BEC = 800
RPT = N // NS

TN = 2000
IDXB = 3200


def _sc_mesh():
    return plsc.VectorSubcoreMesh(core_axis_name="c", subcore_axis_name="s")


def _zero_acc_slice(rows, acc, tbase, nrows):
    def zf(i, _):
        rows[i, :] = jnp.zeros((C,), jnp.float32)
        return 0

    lax.fori_loop(0, nrows, zf, 0)
    nfull = RPT // nrows
    rem = RPT % nrows
    for k in range(nfull):
        pltpu.sync_copy(rows, acc.at[pl.ds(tbase + k * nrows, nrows)])
    if rem:
        pltpu.sync_copy(rows.at[pl.ds(0, rem)],
                        acc.at[pl.ds(tbase + nfull * nrows, rem)])


def _stage_a_body(x16, ei, out, gidx0, gidx1, didx0, didx1, didx2, didx3,
                  rows0, rows1, acc, isem, gsem, ssem0, ssem1):
    sc = lax.axis_index("c")
    tid = lax.axis_index("s")
    tbase = tid * RPT
    gidx = (gidx0, gidx1)
    didx = (didx0, didx1, didx2, didx3)
    rows = (rows0, rows1)
    ssem = (ssem0, ssem1)
    nbu = 63 - sc
    sc_base = sc * (63 * NS * BEA)
    stride = NS * BEA
    _zero_acc_slice(rows0, acc, tbase, BEA)
    plsc.subcore_barrier()

    def run_batch(bi, k):
        b, c = k % 2, k % 4
        gx, dx, rw = gidx[b], didx[c], rows[b]
        off = sc_base + bi * stride + tid * BEA
        pltpu.make_async_copy(ei.at[pl.ds(off, BEA)], gx, isem).wait()
        pltpu.make_async_copy(ei.at[pl.ds(E + off, BEA)], dx, isem).wait()

        @pl.when(bi >= 2)
        def _():
            pltpu.make_async_copy(rw, acc.at[dx], ssem[b]).wait()

        gd = pltpu.async_copy(x16.at[gx], rw, gsem)

        @pl.when(bi < nbu - 1)
        def _():
            off2 = off + stride
            pltpu.async_copy(ei.at[pl.ds(off2, BEA)], gidx[(b + 1) % 2],
                             isem)
            pltpu.async_copy(ei.at[pl.ds(E + off2, BEA)],
                             didx[(c + 1) % 4], isem)

        gd.wait()
        pltpu.async_copy(rw, acc.at[dx], ssem[b], add=True)

    off0 = sc_base + tid * BEA
    pltpu.async_copy(ei.at[pl.ds(off0, BEA)], gidx0, isem)
    pltpu.async_copy(ei.at[pl.ds(E + off0, BEA)], didx0, isem)

    def outer(j, _):
        for k in range(4):
            run_batch(4 * j + k, k)
        return 0

    lax.fori_loop(0, 15, outer, 0)
    run_batch(60, 0)
    run_batch(61, 1)

    @pl.when(sc == 0)
    def _():
        run_batch(62, 2)

    pltpu.make_async_copy(rows0, acc.at[didx0], ssem0).wait()
    pltpu.make_async_copy(rows1, acc.at[didx1], ssem1).wait()
    plsc.subcore_barrier()
    pltpu.sync_copy(acc.at[pl.ds(tbase, RPT)], out.at[sc, pl.ds(tbase, RPT)])


def _stage_c_body(h1f, ei, out, gidx0, gidx1, didx0, didx1, didx2, didx3,
                  rows0, rows1, acc, isem, gsem, ssem0, ssem1):
    sc = lax.axis_index("c")
    tid = lax.axis_index("s")
    tbase = tid * RPT
    ept = E // NS
    ebase = tid * ept
    nb = ept // BEC
    gidx = (gidx0, gidx1)
    didx = (didx0, didx1, didx2, didx3)
    rows = (rows0, rows1)
    ssem = (ssem0, ssem1)

    for cl in range(NCH // NC):
        chunk = sc * (NCH // NC) + cl
        _zero_acc_slice(rows0, acc, tbase, BEC)
        plsc.subcore_barrier()

        def run_batch(bi, k):
            b, c = k % 2, k % 4
            gx, dx, rw = gidx[b], didx[c], rows[b]
            off = ebase + bi * BEC
            pltpu.make_async_copy(ei.at[pl.ds(off, BEC)], gx, isem).wait()
            pltpu.make_async_copy(ei.at[pl.ds(E + off, BEC)], dx, isem).wait()

            def gf(t, _):
                v = gx[pl.ds(t * 16, 16)]
                gx[pl.ds(t * 16, 16)] = v * NCH + chunk
                return 0

            lax.fori_loop(0, BEC // 16, gf, 0)

            @pl.when(bi >= 2)
            def _():
                pltpu.make_async_copy(rw, acc.at[dx], ssem[b]).wait()

            gd = pltpu.async_copy(h1f.at[gx], rw, gsem)

            @pl.when(bi < nb - 1)
            def _():
                off2 = off + BEC
                pltpu.async_copy(ei.at[pl.ds(off2, BEC)],
                                 gidx[(b + 1) % 2], isem)
                pltpu.async_copy(ei.at[pl.ds(E + off2, BEC)],
                                 didx[(c + 1) % 4], isem)

            gd.wait()
            pltpu.async_copy(rw, acc.at[dx], ssem[b], add=True)

        pltpu.async_copy(ei.at[pl.ds(ebase, BEC)], gidx0, isem)
        pltpu.async_copy(ei.at[pl.ds(E + ebase, BEC)], didx0, isem)

        def outer(j, _):
            for k in range(4):
                run_batch(4 * j + k, k)
            return 0

        lax.fori_loop(0, (nb - 1) // 4, outer, 0)
        run_batch(nb - 1, 0)
        pltpu.make_async_copy(rows1, acc.at[didx3], ssem1).wait()
        pltpu.make_async_copy(rows0, acc.at[didx0], ssem0).wait()

        plsc.subcore_barrier()
        pltpu.sync_copy(acc.at[pl.ds(tbase, RPT)],
                        out.at[pl.ds(tbase, RPT), pl.ds(chunk * C, C)])
        plsc.subcore_barrier()


def _run_stage_a(x16, ei):
    f = pl.kernel(
        _stage_a_body,
        out_type=jax.ShapeDtypeStruct((NC, N, C), jnp.float32),
        mesh=_sc_mesh(),
        compiler_params=pltpu.CompilerParams(use_tc_tiling_on_sc=False),
        scratch_types=[
            pltpu.VMEM((BEA,), jnp.int32),
            pltpu.VMEM((BEA,), jnp.int32),
            pltpu.VMEM((BEA,), jnp.int32),
            pltpu.VMEM((BEA,), jnp.int32),
            pltpu.VMEM((BEA,), jnp.int32),
            pltpu.VMEM((BEA,), jnp.int32),
            pltpu.VMEM((BEA, C), jnp.float32),
            pltpu.VMEM((BEA, C), jnp.float32),
            pltpu.VMEM_SHARED((N, C), jnp.float32),
            pltpu.SemaphoreType.DMA,
            pltpu.SemaphoreType.DMA,
            pltpu.SemaphoreType.DMA,
            pltpu.SemaphoreType.DMA,
        ],
    )
    return f(x16, ei)


def _run_stage_c(h1f, ei):
    f = pl.kernel(
        _stage_c_body,
        out_type=jax.ShapeDtypeStruct((N, D), jnp.float32),
        mesh=_sc_mesh(),
        compiler_params=pltpu.CompilerParams(use_tc_tiling_on_sc=False),
        scratch_types=[
            pltpu.VMEM((BEC,), jnp.int32),
            pltpu.VMEM((BEC,), jnp.int32),
            pltpu.VMEM((BEC,), jnp.int32),
            pltpu.VMEM((BEC,), jnp.int32),
            pltpu.VMEM((BEC,), jnp.int32),
            pltpu.VMEM((BEC,), jnp.int32),
            pltpu.VMEM((BEC, C), jnp.float32),
            pltpu.VMEM((BEC, C), jnp.float32),
            pltpu.VMEM_SHARED((N, C), jnp.float32),
            pltpu.SemaphoreType.DMA,
            pltpu.SemaphoreType.DMA,
            pltpu.SemaphoreType.DMA,
            pltpu.SemaphoreType.DMA,
        ],
    )
    return f(h1f, ei)


TNX = 4000


def _x16_body(x_ref, out_ref):
    blk = jnp.concatenate(
        [x_ref[...],
         jnp.ones((TNX, 1), jnp.float32),
         jnp.zeros((TNX, C - 3), jnp.float32)], axis=1)
    out_ref[...] = blk


def _build_x16(x):
    return pl.pallas_call(
        _x16_body,
        grid=(N // TNX,),
        in_specs=[pl.BlockSpec((TNX, 2), lambda i: (i, 0))],
        out_specs=pl.BlockSpec((TNX, C), lambda i: (i, 0)),
        out_shape=jax.ShapeDtypeStruct((N, C), jnp.float32),
    )(x)


def _h1_body(x_ref, agg_ref, w1l_ref, w1r_ref, b1_ref, out_ref):
    s = agg_ref[0] + agg_ref[1]
    cnt = jnp.maximum(s[:, 2:3], 1.0)
    mean1 = s[:, 0:2] / cnt
    h = jnp.dot(x_ref[...], w1l_ref[...], preferred_element_type=jnp.float32)
    h = h + jnp.dot(mean1, w1r_ref[...], preferred_element_type=jnp.float32)
    out_ref[...] = jnp.maximum(h + b1_ref[...], 0.0)


def _run_h1(x, agg1p, W1l, W1r, b1):
    return pl.pallas_call(
        _h1_body,
        grid=(N // TN,),
        in_specs=[
            pl.BlockSpec((TN, 2), lambda i: (i, 0)),
            pl.BlockSpec((NC, TN, C), lambda i: (0, i, 0)),
            pl.BlockSpec((2, D), lambda i: (0, 0)),
            pl.BlockSpec((2, D), lambda i: (0, 0)),
            pl.BlockSpec((1, D), lambda i: (0, 0)),
        ],
        out_specs=pl.BlockSpec((TN, D), lambda i: (i, 0)),
        out_shape=jax.ShapeDtypeStruct((N, D), jnp.float32),
    )(x, agg1p, W1l, W1r, b1.reshape(1, D))


def _dec_body(h1_ref, agg2_ref, agg1_ref, w2l, w2r, b2, wd1, bdd1, wd2, bdd2,
              wd3, bdd3, out_ref):
    s = agg1_ref[0] + agg1_ref[1]
    cnt = jnp.maximum(s[:, 2:3], 1.0)
    mean2 = agg2_ref[...] / cnt
    h1 = h1_ref[...]
    h2 = jnp.dot(h1, w2l[...], preferred_element_type=jnp.float32)
    h2 = h2 + jnp.dot(mean2, w2r[...], preferred_element_type=jnp.float32)
    h2 = h2 + b2[...]
    z = jnp.maximum(
        jnp.dot(h2, wd1[...], preferred_element_type=jnp.float32) + bdd1[...],
        0.0)
    z = jnp.maximum(
        jnp.dot(z, wd2[...], preferred_element_type=jnp.float32) + bdd2[...],
        0.0)
    out_ref[...] = jnp.dot(z, wd3[...],
                           preferred_element_type=jnp.float32) + bdd3[...]


def _run_decoder(h1, agg2, agg1p, W2l, W2r, b2, Wd1, bd1, Wd2, bd2, Wd3, bd3):
    full = lambda shape: pl.BlockSpec(shape, lambda i: tuple(0 for _ in shape))
    return pl.pallas_call(
        _dec_body,
        grid=(N // TN,),
        in_specs=[
            pl.BlockSpec((TN, D), lambda i: (i, 0)),
            pl.BlockSpec((TN, D), lambda i: (i, 0)),
            pl.BlockSpec((NC, TN, C), lambda i: (0, i, 0)),
            full((D, D)),
            full((D, D)),
            full((1, D)),
            full((D, 128)),
            full((1, 128)),
            full((128, 64)),
            full((1, 64)),
            full((64, 1)),
            full((1, 1)),
        ],
        out_specs=pl.BlockSpec((TN, 1), lambda i: (i, 0)),
        out_shape=jax.ShapeDtypeStruct((N, 1), jnp.float32),
    )(h1, agg2, agg1p, W2l, W2r, b2.reshape(1, D), Wd1, bd1.reshape(1, 128),
      Wd2, bd2.reshape(1, 64), Wd3, bd3.reshape(1, 1))


def kernel(x, edge_index, W1l, W1r, b1, W2l, W2r, b2, Wd1, bd1, Wd2, bd2, Wd3,
           bd3):
    ei = edge_index.astype(jnp.int32).reshape(2 * E)
    x16 = _build_x16(x)

    agg1p = _run_stage_a(x16, ei)
    h1 = _run_h1(x, agg1p, W1l, W1r, b1)
    agg2 = _run_stage_c(h1.reshape(N * NCH, C), ei)
    q = _run_decoder(h1, agg2, agg1p, W2l, W2r, b2, Wd1, bd1,
                     Wd2, bd2, Wd3, bd3)
    return q.reshape(N)


# speedup vs baseline: 12.4764x; 1.0767x over previous
import functools

import jax
import jax.numpy as jnp
from jax import lax
from jax.experimental import pallas as pl
from jax.experimental.pallas import tpu as pltpu
from jax.experimental.pallas import tpu_sc as plsc

N = 100000
E = 1600000
D = 128
C = 16
NCH = D // C
NC = 2
NS = 16
BEA = 800
BEC = 800
RPT = N // NS

TN = 2000
IDXB = 3200


def _sc_mesh():
    return plsc.VectorSubcoreMesh(core_axis_name="c", subcore_axis_name="s")


def _zero_acc_slice(rows, acc, tbase, nrows):
    def zf(i, _):
        rows[i, :] = jnp.zeros((C,), jnp.float32)
        return 0

    lax.fori_loop(0, nrows, zf, 0)
    nfull = RPT // nrows
    rem = RPT % nrows
    for k in range(nfull):
        pltpu.sync_copy(rows, acc.at[pl.ds(tbase + k * nrows, nrows)])
    if rem:
        pltpu.sync_copy(rows.at[pl.ds(0, rem)],
                        acc.at[pl.ds(tbase + nfull * nrows, rem)])


def _stage_a_body(x16, ei, out, gidx0, gidx1, didx0, didx1, didx2, didx3,
                  rows0, rows1, acc, isem, gsem, ssem0, ssem1):
    sc = lax.axis_index("c")
    tid = lax.axis_index("s")
    tbase = tid * RPT
    gidx = (gidx0, gidx1)
    didx = (didx0, didx1, didx2, didx3)
    rows = (rows0, rows1)
    ssem = (ssem0, ssem1)
    nbu = 63 - sc
    sc_base = sc * (63 * NS * BEA)
    stride = NS * BEA
    _zero_acc_slice(rows0, acc, tbase, BEA)
    plsc.subcore_barrier()

    def run_batch(bi, k):
        b, c = k % 2, k % 4
        gx, dx, rw = gidx[b], didx[c], rows[b]
        off = sc_base + bi * stride + tid * BEA
        pltpu.make_async_copy(ei.at[pl.ds(off, BEA)], gx, isem).wait()
        pltpu.make_async_copy(ei.at[pl.ds(E + off, BEA)], dx, isem).wait()

        @pl.when(bi >= 2)
        def _():
            pltpu.make_async_copy(rw, acc.at[dx], ssem[b]).wait()

        gd = pltpu.async_copy(x16.at[gx], rw, gsem)

        @pl.when(bi < nbu - 1)
        def _():
            off2 = off + stride
            pltpu.async_copy(ei.at[pl.ds(off2, BEA)], gidx[(b + 1) % 2],
                             isem)
            pltpu.async_copy(ei.at[pl.ds(E + off2, BEA)],
                             didx[(c + 1) % 4], isem)

        gd.wait()
        pltpu.async_copy(rw, acc.at[dx], ssem[b], add=True)

    off0 = sc_base + tid * BEA
    pltpu.async_copy(ei.at[pl.ds(off0, BEA)], gidx0, isem)
    pltpu.async_copy(ei.at[pl.ds(E + off0, BEA)], didx0, isem)

    def outer(j, _):
        for k in range(4):
            run_batch(4 * j + k, k)
        return 0

    lax.fori_loop(0, 15, outer, 0)
    run_batch(60, 0)
    run_batch(61, 1)

    @pl.when(sc == 0)
    def _():
        run_batch(62, 2)

    pltpu.make_async_copy(rows0, acc.at[didx0], ssem0).wait()
    pltpu.make_async_copy(rows1, acc.at[didx1], ssem1).wait()
    plsc.subcore_barrier()
    pltpu.sync_copy(acc.at[pl.ds(tbase, RPT)], out.at[sc, pl.ds(tbase, RPT)])


def _stage_c_body(h1f, ei, out, gidx0, gidx1, didx0, didx1, didx2, didx3,
                  rows0, rows1, acc, isem, gsem, ssem0, ssem1):
    sc = lax.axis_index("c")
    tid = lax.axis_index("s")
    tbase = tid * RPT
    ept = E // NS
    ebase = tid * ept
    nb = ept // BEC
    gidx = (gidx0, gidx1)
    didx = (didx0, didx1, didx2, didx3)
    rows = (rows0, rows1)
    ssem = (ssem0, ssem1)

    for cl in range(NCH // NC):
        chunk = sc * (NCH // NC) + cl
        _zero_acc_slice(rows0, acc, tbase, BEC)
        plsc.subcore_barrier()

        def scale(gx):
            def gf(t, _):
                v = gx[pl.ds(t * 16, 16)]
                gx[pl.ds(t * 16, 16)] = v * NCH + chunk
                return 0

            lax.fori_loop(0, BEC // 16, gf, 0)

        def run_batch(bi, k):
            b, c = k % 2, k % 4
            gx, dx, rw = gidx[b], didx[c], rows[b]
            b2, c2 = (b + 1) % 2, (c + 1) % 4
            off = ebase + bi * BEC

            @pl.when(bi >= 2)
            def _():
                pltpu.make_async_copy(rw, acc.at[dx], ssem[b]).wait()

            gd = pltpu.async_copy(h1f.at[gx], rw, gsem)

            @pl.when(bi < nb - 1)
            def _():
                off2 = off + BEC
                pltpu.async_copy(ei.at[pl.ds(off2, BEC)], gidx[b2], isem)
                pltpu.async_copy(ei.at[pl.ds(E + off2, BEC)], didx[c2],
                                 isem)
                pltpu.make_async_copy(ei.at[pl.ds(off2, BEC)], gidx[b2],
                                      isem).wait()
                pltpu.make_async_copy(ei.at[pl.ds(E + off2, BEC)],
                                      didx[c2], isem).wait()
                scale(gidx[b2])

            gd.wait()
            pltpu.async_copy(rw, acc.at[dx], ssem[b], add=True)

        pltpu.async_copy(ei.at[pl.ds(ebase, BEC)], gidx0, isem)
        pltpu.async_copy(ei.at[pl.ds(E + ebase, BEC)], didx0, isem)
        pltpu.make_async_copy(ei.at[pl.ds(ebase, BEC)], gidx0, isem).wait()
        pltpu.make_async_copy(ei.at[pl.ds(E + ebase, BEC)], didx0,
                              isem).wait()
        scale(gidx0)

        def outer(j, _):
            for k in range(4):
                run_batch(4 * j + k, k)
            return 0

        lax.fori_loop(0, (nb - 1) // 4, outer, 0)
        run_batch(nb - 1, 0)
        pltpu.make_async_copy(rows1, acc.at[didx3], ssem1).wait()
        pltpu.make_async_copy(rows0, acc.at[didx0], ssem0).wait()

        plsc.subcore_barrier()
        pltpu.sync_copy(acc.at[pl.ds(tbase, RPT)],
                        out.at[pl.ds(tbase, RPT), pl.ds(chunk * C, C)])
        plsc.subcore_barrier()


def _run_stage_a(x16, ei):
    f = pl.kernel(
        _stage_a_body,
        out_type=jax.ShapeDtypeStruct((NC, N, C), jnp.float32),
        mesh=_sc_mesh(),
        compiler_params=pltpu.CompilerParams(use_tc_tiling_on_sc=False),
        scratch_types=[
            pltpu.VMEM((BEA,), jnp.int32),
            pltpu.VMEM((BEA,), jnp.int32),
            pltpu.VMEM((BEA,), jnp.int32),
            pltpu.VMEM((BEA,), jnp.int32),
            pltpu.VMEM((BEA,), jnp.int32),
            pltpu.VMEM((BEA,), jnp.int32),
            pltpu.VMEM((BEA, C), jnp.float32),
            pltpu.VMEM((BEA, C), jnp.float32),
            pltpu.VMEM_SHARED((N, C), jnp.float32),
            pltpu.SemaphoreType.DMA,
            pltpu.SemaphoreType.DMA,
            pltpu.SemaphoreType.DMA,
            pltpu.SemaphoreType.DMA,
        ],
    )
    return f(x16, ei)


def _run_stage_c(h1f, ei):
    f = pl.kernel(
        _stage_c_body,
        out_type=jax.ShapeDtypeStruct((N, D), jnp.float32),
        mesh=_sc_mesh(),
        compiler_params=pltpu.CompilerParams(use_tc_tiling_on_sc=False),
        scratch_types=[
            pltpu.VMEM((BEC,), jnp.int32),
            pltpu.VMEM((BEC,), jnp.int32),
            pltpu.VMEM((BEC,), jnp.int32),
            pltpu.VMEM((BEC,), jnp.int32),
            pltpu.VMEM((BEC,), jnp.int32),
            pltpu.VMEM((BEC,), jnp.int32),
            pltpu.VMEM((BEC, C), jnp.float32),
            pltpu.VMEM((BEC, C), jnp.float32),
            pltpu.VMEM_SHARED((N, C), jnp.float32),
            pltpu.SemaphoreType.DMA,
            pltpu.SemaphoreType.DMA,
            pltpu.SemaphoreType.DMA,
            pltpu.SemaphoreType.DMA,
        ],
    )
    return f(h1f, ei)


TNX = 4000


def _x16_body(x_ref, out_ref):
    blk = jnp.concatenate(
        [x_ref[...],
         jnp.ones((TNX, 1), jnp.float32),
         jnp.zeros((TNX, C - 3), jnp.float32)], axis=1)
    out_ref[...] = blk


def _build_x16(x):
    return pl.pallas_call(
        _x16_body,
        grid=(N // TNX,),
        in_specs=[pl.BlockSpec((TNX, 2), lambda i: (i, 0))],
        out_specs=pl.BlockSpec((TNX, C), lambda i: (i, 0)),
        out_shape=jax.ShapeDtypeStruct((N, C), jnp.float32),
    )(x)


def _h1_body(x_ref, agg_ref, w1l_ref, w1r_ref, b1_ref, out_ref):
    s = agg_ref[0] + agg_ref[1]
    cnt = jnp.maximum(s[:, 2:3], 1.0)
    mean1 = s[:, 0:2] / cnt
    h = jnp.dot(x_ref[...], w1l_ref[...], preferred_element_type=jnp.float32)
    h = h + jnp.dot(mean1, w1r_ref[...], preferred_element_type=jnp.float32)
    out_ref[...] = jnp.maximum(h + b1_ref[...], 0.0)


def _run_h1(x, agg1p, W1l, W1r, b1):
    return pl.pallas_call(
        _h1_body,
        grid=(N // TN,),
        in_specs=[
            pl.BlockSpec((TN, 2), lambda i: (i, 0)),
            pl.BlockSpec((NC, TN, C), lambda i: (0, i, 0)),
            pl.BlockSpec((2, D), lambda i: (0, 0)),
            pl.BlockSpec((2, D), lambda i: (0, 0)),
            pl.BlockSpec((1, D), lambda i: (0, 0)),
        ],
        out_specs=pl.BlockSpec((TN, D), lambda i: (i, 0)),
        out_shape=jax.ShapeDtypeStruct((N, D), jnp.float32),
    )(x, agg1p, W1l, W1r, b1.reshape(1, D))


def _dec_body(h1_ref, agg2_ref, agg1_ref, w2l, w2r, b2, wd1, bdd1, wd2, bdd2,
              wd3, bdd3, out_ref):
    s = agg1_ref[0] + agg1_ref[1]
    cnt = jnp.maximum(s[:, 2:3], 1.0)
    mean2 = agg2_ref[...] / cnt
    h1 = h1_ref[...]
    h2 = jnp.dot(h1, w2l[...], preferred_element_type=jnp.float32)
    h2 = h2 + jnp.dot(mean2, w2r[...], preferred_element_type=jnp.float32)
    h2 = h2 + b2[...]
    z = jnp.maximum(
        jnp.dot(h2, wd1[...], preferred_element_type=jnp.float32) + bdd1[...],
        0.0)
    z = jnp.maximum(
        jnp.dot(z, wd2[...], preferred_element_type=jnp.float32) + bdd2[...],
        0.0)
    out_ref[...] = jnp.dot(z, wd3[...],
                           preferred_element_type=jnp.float32) + bdd3[...]


def _run_decoder(h1, agg2, agg1p, W2l, W2r, b2, Wd1, bd1, Wd2, bd2, Wd3, bd3):
    full = lambda shape: pl.BlockSpec(shape, lambda i: tuple(0 for _ in shape))
    return pl.pallas_call(
        _dec_body,
        grid=(N // TN,),
        in_specs=[
            pl.BlockSpec((TN, D), lambda i: (i, 0)),
            pl.BlockSpec((TN, D), lambda i: (i, 0)),
            pl.BlockSpec((NC, TN, C), lambda i: (0, i, 0)),
            full((D, D)),
            full((D, D)),
            full((1, D)),
            full((D, 128)),
            full((1, 128)),
            full((128, 64)),
            full((1, 64)),
            full((64, 1)),
            full((1, 1)),
        ],
        out_specs=pl.BlockSpec((TN, 1), lambda i: (i, 0)),
        out_shape=jax.ShapeDtypeStruct((N, 1), jnp.float32),
    )(h1, agg2, agg1p, W2l, W2r, b2.reshape(1, D), Wd1, bd1.reshape(1, 128),
      Wd2, bd2.reshape(1, 64), Wd3, bd3.reshape(1, 1))


def kernel(x, edge_index, W1l, W1r, b1, W2l, W2r, b2, Wd1, bd1, Wd2, bd2, Wd3,
           bd3):
    ei = edge_index.astype(jnp.int32).reshape(2 * E)
    x16 = _build_x16(x)

    agg1p = _run_stage_a(x16, ei)
    h1 = _run_h1(x, agg1p, W1l, W1r, b1)
    agg2 = _run_stage_c(h1.reshape(N * NCH, C), ei)
    q = _run_decoder(h1, agg2, agg1p, W2l, W2r, b2, Wd1, bd1,
                     Wd2, bd2, Wd3, bd3)
    return q.reshape(N)
